# initial kernel scaffold (unmeasured)
import jax
import jax.numpy as jnp
from jax import lax
from jax.experimental import pallas as pl
from jax.experimental.pallas import tpu as pltpu


def kernel(
    x,
):
    def body(*refs):
        pass

    out_shape = jax.ShapeDtypeStruct(..., jnp.float32)
    return pl.pallas_call(body, out_shape=out_shape)(...)


# baseline (device time: 600075 ns/iter reference)
import jax
import jax.numpy as jnp
from jax import lax
from jax.experimental import pallas as pl
from jax.experimental.pallas import tpu as pltpu

N_DEV = 4
N_HOPS = N_DEV - 1


def kernel(x):
    m, n = x.shape
    ch = m // N_DEV

    def body(x_ref, out_ref, comm_ref, send_sems, recv_sems, copy_sem):
        my = lax.axis_index("i")
        left = (my + N_DEV - 1) % N_DEV
        right = (my + 1) % N_DEV

        barrier_sem = pltpu.get_barrier_semaphore()
        for nbr in (left, right):
            pl.semaphore_signal(
                barrier_sem, inc=1,
                device_id=(nbr,), device_id_type=pl.DeviceIdType.MESH,
            )
        pl.semaphore_wait(barrier_sem, 2)

        load = pltpu.make_async_copy(x_ref, out_ref, copy_sem)
        load.start()
        load.wait()

        for s in range(N_HOPS):
            send_c = (my - s + N_DEV) % N_DEV
            recv_c = (my - s - 1 + 2 * N_DEV) % N_DEV
            rdma = pltpu.make_async_remote_copy(
                src_ref=out_ref.at[pl.ds(send_c * ch, ch), :],
                dst_ref=comm_ref.at[s],
                send_sem=send_sems.at[s],
                recv_sem=recv_sems.at[s],
                device_id=(right,),
                device_id_type=pl.DeviceIdType.MESH,
            )
            rdma.start()
            rdma.wait()
            out_ref[pl.ds(recv_c * ch, ch), :] += comm_ref[s]

        for s in range(N_HOPS):
            send_c = (my + 1 - s + N_DEV) % N_DEV
            recv_c = (my - s + N_DEV) % N_DEV
            send = pltpu.make_async_remote_copy(
                src_ref=out_ref.at[pl.ds(send_c * ch, ch), :],
                dst_ref=out_ref.at[pl.ds(send_c * ch, ch), :],
                send_sem=send_sems.at[N_HOPS + s],
                recv_sem=recv_sems.at[N_HOPS + s],
                device_id=(right,),
                device_id_type=pl.DeviceIdType.MESH,
            )
            send.start()
            send.wait_send()
            recv = pltpu.make_async_remote_copy(
                src_ref=out_ref.at[pl.ds(recv_c * ch, ch), :],
                dst_ref=out_ref.at[pl.ds(recv_c * ch, ch), :],
                send_sem=send_sems.at[N_HOPS + s],
                recv_sem=recv_sems.at[N_HOPS + s],
                device_id=(left,),
                device_id_type=pl.DeviceIdType.MESH,
            )
            recv.wait_recv()

    return pl.pallas_call(
        body,
        out_shape=jax.ShapeDtypeStruct((m, n), x.dtype),
        in_specs=[pl.BlockSpec(memory_space=pl.ANY)],
        out_specs=pl.BlockSpec(memory_space=pltpu.VMEM),
        scratch_shapes=[
            pltpu.VMEM((N_HOPS, ch, n), x.dtype),
            pltpu.SemaphoreType.DMA((2 * N_HOPS,)),
            pltpu.SemaphoreType.DMA((2 * N_HOPS,)),
            pltpu.SemaphoreType.DMA,
        ],
        compiler_params=pltpu.CompilerParams(
            collective_id=0,
            vmem_limit_bytes=63 * 1024 * 1024,
        ),
    )(x)


# device time: 329817 ns/iter; 1.8194x vs baseline; 1.8194x over previous
import jax
import jax.numpy as jnp
from jax import lax
from jax.experimental import pallas as pl
from jax.experimental.pallas import tpu as pltpu

N_DEV = 4
N_HOPS = N_DEV - 1
PLUS, MINUS = 0, 1


def kernel(x):
    m, n = x.shape
    ch = m // N_DEV
    n2 = n // 2

    def body(x_ref, out_ref, comm_ref, send_sems, recv_sems, copy_sem):
        my = lax.axis_index("i")
        left = (my + N_DEV - 1) % N_DEV
        right = (my + 1) % N_DEV

        barrier_sem = pltpu.get_barrier_semaphore()
        for nbr in (left, right):
            pl.semaphore_signal(
                barrier_sem, inc=1,
                device_id=(nbr,), device_id_type=pl.DeviceIdType.MESH,
            )
        pl.semaphore_wait(barrier_sem, 2)

        load = pltpu.make_async_copy(x_ref, out_ref, copy_sem)
        load.start()
        load.wait()

        def ring_rdma(dirn, src_rows, src_cols, dst_ref, hop, peer):
            return pltpu.make_async_remote_copy(
                src_ref=out_ref.at[src_rows, src_cols],
                dst_ref=dst_ref,
                send_sem=send_sems.at[dirn, hop],
                recv_sem=recv_sems.at[dirn, hop],
                device_id=(peer,),
                device_id_type=pl.DeviceIdType.MESH,
            )

        cols_p = pl.ds(0, n2)
        cols_m = pl.ds(n2, n2)

        for s in range(N_HOPS):
            sp = (my - s + N_DEV) % N_DEV
            rp = (my - s - 1 + 2 * N_DEV) % N_DEV
            sm = (my + s) % N_DEV
            rm = (my + s + 1) % N_DEV
            rd_p = ring_rdma(PLUS, pl.ds(sp * ch, ch), cols_p,
                             comm_ref.at[PLUS, s], s, right)
            rd_m = ring_rdma(MINUS, pl.ds(sm * ch, ch), cols_m,
                             comm_ref.at[MINUS, s], s, left)
            rd_p.start()
            rd_m.start()
            rd_p.wait()
            rd_m.wait()
            out_ref[pl.ds(rp * ch, ch), cols_p] += comm_ref[PLUS, s]
            out_ref[pl.ds(rm * ch, ch), cols_m] += comm_ref[MINUS, s]


        for s in range(N_HOPS):
            sp = (my + 1 - s + N_DEV) % N_DEV
            rp = (my - s + N_DEV) % N_DEV
            sm = (my - 1 + s + N_DEV) % N_DEV
            rm = (my + s) % N_DEV
            h = N_HOPS + s
            snd_p = ring_rdma(PLUS, pl.ds(sp * ch, ch), cols_p,
                              out_ref.at[pl.ds(sp * ch, ch), cols_p], h, right)
            snd_m = ring_rdma(MINUS, pl.ds(sm * ch, ch), cols_m,
                              out_ref.at[pl.ds(sm * ch, ch), cols_m], h, left)
            snd_p.start()
            snd_m.start()
            snd_p.wait_send()
            snd_m.wait_send()
            rcv_p = ring_rdma(PLUS, pl.ds(rp * ch, ch), cols_p,
                              out_ref.at[pl.ds(rp * ch, ch), cols_p], h, left)
            rcv_m = ring_rdma(MINUS, pl.ds(rm * ch, ch), cols_m,
                              out_ref.at[pl.ds(rm * ch, ch), cols_m], h, right)
            rcv_p.wait_recv()
            rcv_m.wait_recv()

    return pl.pallas_call(
        body,
        out_shape=jax.ShapeDtypeStruct((m, n), x.dtype),
        in_specs=[pl.BlockSpec(memory_space=pl.ANY)],
        out_specs=pl.BlockSpec(memory_space=pltpu.VMEM),
        scratch_shapes=[
            pltpu.VMEM((2, N_HOPS, ch, n2), x.dtype),
            pltpu.SemaphoreType.DMA((2, 2 * N_HOPS)),
            pltpu.SemaphoreType.DMA((2, 2 * N_HOPS)),
            pltpu.SemaphoreType.DMA,
        ],
        compiler_params=pltpu.CompilerParams(
            collective_id=0,
            vmem_limit_bytes=63 * 1024 * 1024,
        ),
    )(x)


# device time: 322467 ns/iter; 1.8609x vs baseline; 1.0228x over previous
import jax
import jax.numpy as jnp
from jax import lax
from jax.experimental import pallas as pl
from jax.experimental.pallas import tpu as pltpu

N_DEV = 4
N_HOPS = N_DEV - 1
PLUS, MINUS = 0, 1


def kernel(x):
    m, n = x.shape
    ch = m // N_DEV
    n2 = n // 2

    def body(x_ref, out_ref, comm_ref, send_sems, recv_sems, copy_sems):
        my = lax.axis_index("i")
        left = (my + N_DEV - 1) % N_DEV
        right = (my + 1) % N_DEV

        def load_chunk(k):
            c = (my - k + N_DEV) % N_DEV
            rows = pl.ds(c * ch, ch)
            cp = pltpu.make_async_copy(
                x_ref.at[rows], out_ref.at[rows], copy_sems.at[k]
            )
            cp.start()
            return cp

        loads = [load_chunk(k) for k in range(N_DEV)]

        barrier_sem = pltpu.get_barrier_semaphore()
        for nbr in (left, right):
            pl.semaphore_signal(
                barrier_sem, inc=1,
                device_id=(nbr,), device_id_type=pl.DeviceIdType.MESH,
            )
        pl.semaphore_wait(barrier_sem, 2)

        loads[0].wait()

        def ring_rdma(dirn, src_rows, src_cols, dst_ref, hop, peer):
            return pltpu.make_async_remote_copy(
                src_ref=out_ref.at[src_rows, src_cols],
                dst_ref=dst_ref,
                send_sem=send_sems.at[dirn, hop],
                recv_sem=recv_sems.at[dirn, hop],
                device_id=(peer,),
                device_id_type=pl.DeviceIdType.MESH,
            )

        cols_p = pl.ds(0, n2)
        cols_m = pl.ds(n2, n2)

        loaded = {0}
        for s in range(N_HOPS):
            sp = (my - s + N_DEV) % N_DEV
            rp = (my - s - 1 + 2 * N_DEV) % N_DEV
            sm = (my + s) % N_DEV
            rm = (my + s + 1) % N_DEV
            rd_p = ring_rdma(PLUS, pl.ds(sp * ch, ch), cols_p,
                             comm_ref.at[PLUS, s], s, right)
            rd_m = ring_rdma(MINUS, pl.ds(sm * ch, ch), cols_m,
                             comm_ref.at[MINUS, s], s, left)
            rd_p.start()
            rd_m.start()
            for k in sorted({s + 1, N_DEV - 1 - s} - loaded):
                loads[k].wait()
                loaded.add(k)
            rd_p.wait()
            rd_m.wait()
            out_ref[pl.ds(rp * ch, ch), cols_p] += comm_ref[PLUS, s]
            out_ref[pl.ds(rm * ch, ch), cols_m] += comm_ref[MINUS, s]


        for s in range(N_HOPS):
            sp = (my + 1 - s + N_DEV) % N_DEV
            rp = (my - s + N_DEV) % N_DEV
            sm = (my - 1 + s + N_DEV) % N_DEV
            rm = (my + s) % N_DEV
            h = N_HOPS + s
            snd_p = ring_rdma(PLUS, pl.ds(sp * ch, ch), cols_p,
                              out_ref.at[pl.ds(sp * ch, ch), cols_p], h, right)
            snd_m = ring_rdma(MINUS, pl.ds(sm * ch, ch), cols_m,
                              out_ref.at[pl.ds(sm * ch, ch), cols_m], h, left)
            snd_p.start()
            snd_m.start()
            snd_p.wait_send()
            snd_m.wait_send()
            rcv_p = ring_rdma(PLUS, pl.ds(rp * ch, ch), cols_p,
                              out_ref.at[pl.ds(rp * ch, ch), cols_p], h, left)
            rcv_m = ring_rdma(MINUS, pl.ds(rm * ch, ch), cols_m,
                              out_ref.at[pl.ds(rm * ch, ch), cols_m], h, right)
            rcv_p.wait_recv()
            rcv_m.wait_recv()

    return pl.pallas_call(
        body,
        out_shape=jax.ShapeDtypeStruct((m, n), x.dtype),
        in_specs=[pl.BlockSpec(memory_space=pl.ANY)],
        out_specs=pl.BlockSpec(memory_space=pltpu.VMEM),
        scratch_shapes=[
            pltpu.VMEM((2, N_HOPS, ch, n2), x.dtype),
            pltpu.SemaphoreType.DMA((2, 2 * N_HOPS)),
            pltpu.SemaphoreType.DMA((2, 2 * N_HOPS)),
            pltpu.SemaphoreType.DMA((N_DEV,)),
        ],
        compiler_params=pltpu.CompilerParams(
            collective_id=0,
            vmem_limit_bytes=63 * 1024 * 1024,
        ),
    )(x)


# device time: 311412 ns/iter; 1.9269x vs baseline; 1.0355x over previous
import jax
import jax.numpy as jnp
from jax import lax
from jax.experimental import pallas as pl
from jax.experimental.pallas import tpu as pltpu

N_DEV = 4
N_HOPS = N_DEV - 1
N_SUB = 2
PLUS, MINUS = 0, 1


def kernel(x):
    m, n = x.shape
    ch = m // N_DEV
    ch2 = ch // N_SUB
    n2 = n // 2

    def body(x_ref, out_ref, comm_ref, send_sems, recv_sems, copy_sems):
        my = lax.axis_index("i")
        left = (my + N_DEV - 1) % N_DEV
        right = (my + 1) % N_DEV

        def load_chunk(k):
            c = (my - k + N_DEV) % N_DEV
            rows_k = pl.ds(c * ch, ch)
            cp = pltpu.make_async_copy(
                x_ref.at[rows_k], out_ref.at[rows_k], copy_sems.at[k]
            )
            cp.start()
            return cp

        loads = [load_chunk(k) for k in range(N_DEV)]

        barrier_sem = pltpu.get_barrier_semaphore()
        for nbr in (left, right):
            pl.semaphore_signal(
                barrier_sem, inc=1,
                device_id=(nbr,), device_id_type=pl.DeviceIdType.MESH,
            )
        pl.semaphore_wait(barrier_sem, 2)

        loads[0].wait()

        cols = {PLUS: pl.ds(0, n2), MINUS: pl.ds(n2, n2)}
        peer_out = {PLUS: right, MINUS: left}
        peer_in = {PLUS: left, MINUS: right}

        def rows(c, t):
            return pl.ds(c * ch + t * ch2, ch2)

        def rchunk(dirn, h):
            step = h + 1 if h < N_HOPS else h - N_HOPS
            if dirn == PLUS:
                return (my - step + N_DEV) % N_DEV
            return (my + step) % N_DEV

        def schunk(dirn, h):
            return my if h == 0 else rchunk(dirn, h - 1)

        def mk(dirn, h, t, is_send):
            c = schunk(dirn, h) if is_send else rchunk(dirn, h)
            if h < N_HOPS:
                buf = comm_ref.at[dirn, h, t]
            else:
                buf = out_ref.at[rows(c, t), cols[dirn]]
            return pltpu.make_async_remote_copy(
                src_ref=out_ref.at[rows(c, t), cols[dirn]],
                dst_ref=buf,
                send_sem=send_sems.at[dirn, h, t],
                recv_sem=recv_sems.at[dirn, h, t],
                device_id=(peer_out[dirn] if is_send else peer_in[dirn],),
                device_id_type=pl.DeviceIdType.MESH,
            )

        for t in range(N_SUB):
            for d in (PLUS, MINUS):
                mk(d, 0, t, True).start()

        loaded = {0}
        for h in range(2 * N_HOPS):
            if h < N_HOPS:
                for k in sorted({h + 1, N_DEV - 1 - h} - loaded):
                    loads[k].wait()
                    loaded.add(k)
            for t in range(N_SUB):
                for d in (PLUS, MINUS):
                    mk(d, h, t, False).wait_recv()
                    if h < N_HOPS:
                        out_ref[rows(rchunk(d, h), t), cols[d]] += (
                            comm_ref[d, h, t]
                        )
                    if h + 1 < 2 * N_HOPS:
                        mk(d, h + 1, t, True).start()

        for h in range(2 * N_HOPS):
            for t in range(N_SUB):
                for d in (PLUS, MINUS):
                    mk(d, h, t, True).wait_send()

    return pl.pallas_call(
        body,
        out_shape=jax.ShapeDtypeStruct((m, n), x.dtype),
        in_specs=[pl.BlockSpec(memory_space=pl.ANY)],
        out_specs=pl.BlockSpec(memory_space=pltpu.VMEM),
        scratch_shapes=[
            pltpu.VMEM((2, N_HOPS, N_SUB, ch2, n2), x.dtype),
            pltpu.SemaphoreType.DMA((2, 2 * N_HOPS, N_SUB)),
            pltpu.SemaphoreType.DMA((2, 2 * N_HOPS, N_SUB)),
            pltpu.SemaphoreType.DMA((N_DEV,)),
        ],
        compiler_params=pltpu.CompilerParams(
            collective_id=0,
            vmem_limit_bytes=63 * 1024 * 1024,
        ),
    )(x)
